# baseline (speedup 1.0000x reference)
import functools
import jax
import jax.numpy as jnp
from jax import lax
from jax.experimental import pallas as pl
from jax.experimental.pallas import tpu as pltpu
from jax.experimental.pallas import tpu_sc as plsc

N = 10000
D = 128
E = 320000
EL = 320000
NC = 2
NS = 16
L = 16
CH = 128
NP = 10240
NCHN = NP // CH
ECH = E // CH
LCH = EL // CH
ECHP = 2560
CPT = ECHP // NS
WPT = ECHP // (NC * NS)
WSUM = 5.0 / 6.0

_MESH = plsc.VectorSubcoreMesh(core_axis_name="c", subcore_axis_name="s")


def _vrsqrt(x):
    i = lax.bitcast_convert_type(x, jnp.int32)
    i = jnp.int32(0x5F3759DF) - lax.shift_right_logical(i, 1)
    y = lax.bitcast_convert_type(i, jnp.float32)
    for _ in range(3):
        y = y * (1.5 - 0.5 * x * y * y)
    return jnp.where(x > 0.0, y, 0.0)


def _splat_elem(ref, r):
    idx = jnp.broadcast_to(r, (L,)).astype(jnp.int32)
    return plsc.load_gather(ref, [idx])


def _fill(ref, n, value):
    for g in range(n // L):
        ref[pl.ds(g * L, L)] = jnp.full((L,), value, jnp.float32)


@functools.partial(
    pl.kernel,
    out_type=[
        jax.ShapeDtypeStruct((NC, 1, NP), jnp.float32),
        jax.ShapeDtypeStruct((NC, NP, D), jnp.float32),
    ],
    mesh=_MESH,
    compiler_params=pltpu.CompilerParams(needs_layout_passes=False),
    scratch_types=[
        pltpu.VMEM_SHARED((NP,), jnp.float32),
        pltpu.VMEM((CPT, CH), jnp.int32),
        pltpu.VMEM((1, CH), jnp.float32),
        pltpu.VMEM((CH,), jnp.float32),
        pltpu.VMEM((CH, D), jnp.float32),
    ],
)
def _phase_a(idxr, emb2, dinv2, xp2, deg_s, idx_b, ones_b, dinv_v, row_v):
    cid = lax.axis_index("c")
    sid = lax.axis_index("s")

    _fill(dinv_v, CH, 0.0)
    for i in range(NCHN // NS):
        base = (sid + i * NS) * CH
        pltpu.sync_copy(dinv_v, deg_s.at[pl.ds(base, CH)])

    for g in range(CH // L):
        ones_b[0, pl.ds(g * L, L)] = jnp.full((L,), 1.0, jnp.float32)
    plsc.subcore_barrier()

    LAST = ECH - (NS - 1) * CPT

    @pl.when(sid < NS - 1)
    def _():
        pltpu.sync_copy(idxr.at[cid, pl.ds(sid * CPT, CPT), :], idx_b)

    @pl.when(sid == NS - 1)
    def _():
        pltpu.sync_copy(idxr.at[cid, pl.ds((NS - 1) * CPT, LAST), :],
                        idx_b.at[pl.ds(0, LAST), :])

    n = jnp.minimum(CPT, ECH - sid * CPT)

    def hist_body(c, carry):
        pltpu.sync_copy(ones_b.at[0], deg_s.at[idx_b.at[c]], add=True)
        return carry

    lax.fori_loop(0, n, hist_body, 0)
    plsc.subcore_barrier()

    for i in range(NCHN // NS):
        base = (sid + i * NS) * CH
        pltpu.sync_copy(deg_s.at[pl.ds(base, CH)], dinv_v)
        for g in range(CH // L):
            dinv_v[pl.ds(g * L, L)] = _vrsqrt(dinv_v[pl.ds(g * L, L)])
        pltpu.sync_copy(dinv_v, dinv2.at[cid, 0, pl.ds(base, CH)])
        pltpu.sync_copy(emb2.at[cid, pl.ds(base, CH), :], row_v)

        def row_body(r, rcarry):
            s = _splat_elem(dinv_v, r)
            for g in range(D // L):
                row_v[r, pl.ds(g * L, L)] = row_v[r, pl.ds(g * L, L)] * s
            return rcarry

        lax.fori_loop(0, CH, row_body, 0)
        pltpu.sync_copy(row_v, xp2.at[cid, pl.ds(base, CH), :])


@functools.partial(
    pl.kernel,
    out_type=jax.ShapeDtypeStruct((NC, NP, D), jnp.float32),
    mesh=_MESH,
    compiler_params=pltpu.CompilerParams(needs_layout_passes=False),
    scratch_types=[
        pltpu.VMEM_SHARED((NP, D), jnp.float32),
        pltpu.VMEM((4, CH), jnp.int32),
        pltpu.VMEM((4, CH), jnp.int32),
        pltpu.VMEM((2, CH, D), jnp.float32),
        pltpu.VMEM((CH,), jnp.float32),
        pltpu.SemaphoreType.DMA,
        pltpu.SemaphoreType.DMA,
        pltpu.SemaphoreType.DMA,
        pltpu.SemaphoreType.DMA,
        pltpu.SemaphoreType.DMA,
        pltpu.SemaphoreType.DMA,
        pltpu.SemaphoreType.DMA,
        pltpu.SemaphoreType.DMA,
    ],
)
def _phase_b(tbl2, idx2, emb2, dinv2, res2,
             acc_s, idxg_r, idxs_r, rows2, dinv_v,
             sem_a, sem_b, sem_i0, sem_i1, sem_i2, sem_i3,
             sem_s0, sem_s1):
    cid = lax.axis_index("c")
    sid = lax.axis_index("s")
    ocid = 1 - cid

    def zrow_body(r, carry):
        for g in range(D // L):
            rows2[0, r, pl.ds(g * L, L)] = jnp.full((L,), 0.0, jnp.float32)
        return carry

    lax.fori_loop(0, CH, zrow_body, 0)
    for i in range(NCHN // NS):
        base = (sid + i * NS) * CH
        pltpu.sync_copy(rows2.at[0], acc_s.at[pl.ds(base, CH), :])
    plsc.subcore_barrier()

    cbase = sid * CPT
    n = jnp.minimum(CPT, ECH - cbase)
    isems = (sem_i0, sem_i1, sem_i2, sem_i3)

    def istart(c, k):
        pltpu.async_copy(
            idx2.at[ocid, 0, pl.ds((cbase + c) * CH, CH)],
            idxg_r.at[k], isems[k])
        pltpu.async_copy(
            idx2.at[cid, 0, pl.ds((cbase + c) * CH, CH)],
            idxs_r.at[k], isems[k])

    def iwait(c, k):
        pltpu.make_async_copy(
            idx2.at[ocid, 0, pl.ds((cbase + c) * CH, CH)],
            idxg_r.at[k], isems[k]).wait()
        pltpu.make_async_copy(
            idx2.at[cid, 0, pl.ds((cbase + c) * CH, CH)],
            idxs_r.at[k], isems[k]).wait()

    def gstart(c, k, rs):
        pltpu.async_copy(
            tbl2.at[ocid].at[idxg_r.at[k]], rows2.at[rs],
            (sem_a, sem_b)[rs])

    def gwait(c, k, rs):
        pltpu.make_async_copy(
            tbl2.at[ocid].at[idxg_r.at[k]], rows2.at[rs],
            (sem_a, sem_b)[rs]).wait()

    ssems = (sem_s0, sem_s1)

    def sstart(k, rs):
        pltpu.async_copy(rows2.at[rs], acc_s.at[idxs_r.at[k]],
                         ssems[rs], add=True)

    def swait(k, rs):
        pltpu.make_async_copy(rows2.at[rs], acc_s.at[idxs_r.at[k]],
                              ssems[rs]).wait()

    for k in range(3):
        istart(k, k)
    iwait(0, 0)
    gstart(0, 0, 0)

    def quad_body(i, carry):
        c0 = 4 * i
        for k in range(4):
            c = c0 + k
            rs = k % 2
            pk = (k - 1) % 4
            gwait(c, k, rs)
            sstart(k, rs)

            @pl.when(c >= 1)
            def _():
                swait(pk, 1 - rs)

            @pl.when(c + 3 < n)
            def _():
                istart(c + 3, pk)

            @pl.when(c + 1 < n)
            def _():
                iwait(c + 1, (k + 1) % 4)
                gstart(c + 1, (k + 1) % 4, 1 - rs)

        return carry

    lax.fori_loop(0, n // 4, quad_body, 0)
    swait(3, 1)
    plsc.subcore_barrier()

    for i in range(NCHN // NS):
        base = (sid + i * NS) * CH
        pltpu.sync_copy(acc_s.at[pl.ds(base, CH), :], rows2.at[0])
        pltpu.sync_copy(dinv2.at[cid, 0, pl.ds(base, CH)], dinv_v)
        pltpu.sync_copy(emb2.at[cid, pl.ds(base, CH), :], rows2.at[1])

        def row_body(r, rcarry):
            w = _splat_elem(dinv_v, r) * WSUM
            for g in range(D // L):
                rows2[1, r, pl.ds(g * L, L)] = (
                    rows2[1, r, pl.ds(g * L, L)]
                    + rows2[0, r, pl.ds(g * L, L)] * w
                )
            return rcarry

        lax.fori_loop(0, CH, row_body, 0)
        pltpu.sync_copy(rows2.at[1], res2.at[cid, pl.ds(base, CH), :])


@functools.partial(
    pl.kernel,
    out_type=jax.ShapeDtypeStruct((EL,), jnp.float32),
    mesh=_MESH,
    compiler_params=pltpu.CompilerParams(needs_layout_passes=False),
    scratch_types=[
        pltpu.VMEM((WPT, CH), jnp.int32),
        pltpu.VMEM((WPT, CH), jnp.int32),
        pltpu.VMEM((2, CH, D), jnp.float32),
        pltpu.VMEM((2, CH, D), jnp.float32),
        pltpu.VMEM((2 * CH,), jnp.float32),
        pltpu.VMEM((L * 17,), jnp.float32),
        pltpu.SemaphoreType.DMA,
        pltpu.SemaphoreType.DMA,
        pltpu.SemaphoreType.DMA,
        pltpu.SemaphoreType.DMA,
    ],
)
def _phase_c(res2, iup, imp, out,
             iu_v, im_v, u2, m2, out_v, p_buf, semu0, semu1, semm0, semm1):
    cid = lax.axis_index("c")
    sid = lax.axis_index("s")
    wid = sid * NC + cid
    pltpu.sync_copy(iup.at[pl.ds(wid * WPT, WPT), :], iu_v)
    pltpu.sync_copy(imp.at[pl.ds(wid * WPT, WPT), :], im_v)
    n = jnp.minimum(WPT, LCH - wid * WPT)
    semu = (semu0, semu1)
    semm = (semm0, semm1)
    lane = lax.iota(jnp.int32, L)

    def cstart(c, slot):
        pltpu.async_copy(res2.at[0].at[iu_v.at[c]], u2.at[slot], semu[slot])
        pltpu.async_copy(res2.at[1].at[im_v.at[c]], m2.at[slot], semm[slot])

    def cwait(c, slot):
        pltpu.make_async_copy(
            res2.at[0].at[iu_v.at[c]], u2.at[slot], semu[slot]).wait()
        pltpu.make_async_copy(
            res2.at[1].at[im_v.at[c]], m2.at[slot], semm[slot]).wait()

    cols = [lane * 17 + l for l in range(L)]

    def dots(slot, par):
        def group_body(g, carry):
            for ep in range(L):
                e = g * L + ep
                p = u2[slot, e, pl.ds(0, L)] * m2[slot, e, pl.ds(0, L)]
                for k in range(1, D // L):
                    p = p + (u2[slot, e, pl.ds(k * L, L)]
                             * m2[slot, e, pl.ds(k * L, L)])
                p_buf[pl.ds(ep * 17, L)] = p
            acc = plsc.load_gather(p_buf, [cols[0]])
            for l in range(1, L):
                acc = acc + plsc.load_gather(p_buf, [cols[l]])
            out_v[pl.ds(par * CH + g * L, L)] = acc
            return carry

        lax.fori_loop(0, CH // L, group_body, 0)

    cstart(0, 0)

    def pair_body(i, carry):
        c0 = 2 * i
        cstart(c0 + 1, 1)
        cwait(c0, 0)
        dots(0, 0)

        @pl.when(c0 + 2 < n)
        def _():
            cstart(c0 + 2, 0)

        cwait(c0 + 1, 1)
        dots(1, 1)
        pltpu.sync_copy(out_v, out.at[pl.ds((wid * WPT + c0) * CH, 2 * CH)])
        return carry

    lax.fori_loop(0, n // 2, pair_body, 0)


def kernel(user_node_id, movie_node_id, edge_index, edge_label_index,
           emb_user, emb_movie):
    del user_node_id, movie_node_id
    emb2 = jnp.zeros((NC, NP, D), jnp.float32)
    emb2 = emb2.at[0, :N, :].set(emb_user).at[1, :N, :].set(emb_movie)
    idx2 = edge_index.reshape(NC, 1, E)
    idxr = edge_index.reshape(NC, ECH, CH)
    dinv2, xp2 = _phase_a(idxr, emb2)
    res2 = _phase_b(xp2, idx2, emb2, dinv2)
    lab = jnp.zeros((2, ECHP * CH), jnp.int32)
    lab = lab.at[:, :EL].set(edge_label_index).reshape(2, ECHP, CH)
    res = _phase_c(res2, lab[1], lab[0])
    return res, res2[0, :N, :], res2[1, :N, :]


# speedup vs baseline: 1.0947x; 1.0947x over previous
import functools
import jax
import jax.numpy as jnp
from jax import lax
from jax.experimental import pallas as pl
from jax.experimental.pallas import tpu as pltpu
from jax.experimental.pallas import tpu_sc as plsc

N = 10000
D = 128
E = 320000
EL = 320000
NC = 2
NS = 16
L = 16
CH = 128
NP = 10240
NCHN = NP // CH
ECH = E // CH
LCH = EL // CH
ECHP = 2560
CPT = ECHP // NS
WPT = ECHP // (NC * NS)
WSUM = 5.0 / 6.0

_MESH = plsc.VectorSubcoreMesh(core_axis_name="c", subcore_axis_name="s")


def _vrsqrt(x):
    i = lax.bitcast_convert_type(x, jnp.int32)
    i = jnp.int32(0x5F3759DF) - lax.shift_right_logical(i, 1)
    y = lax.bitcast_convert_type(i, jnp.float32)
    for _ in range(3):
        y = y * (1.5 - 0.5 * x * y * y)
    return jnp.where(x > 0.0, y, 0.0)


def _splat_elem(ref, r):
    idx = jnp.broadcast_to(r, (L,)).astype(jnp.int32)
    return plsc.load_gather(ref, [idx])


def _fill(ref, n, value):
    for g in range(n // L):
        ref[pl.ds(g * L, L)] = jnp.full((L,), value, jnp.float32)


@functools.partial(
    pl.kernel,
    out_type=[
        jax.ShapeDtypeStruct((NC, 1, NP), jnp.float32),
        jax.ShapeDtypeStruct((NC, NP, D), jnp.float32),
    ],
    mesh=_MESH,
    compiler_params=pltpu.CompilerParams(needs_layout_passes=False),
    scratch_types=[
        pltpu.VMEM_SHARED((NP,), jnp.float32),
        pltpu.VMEM((CPT, CH), jnp.int32),
        pltpu.VMEM((1, CH), jnp.float32),
        pltpu.VMEM((CH,), jnp.float32),
        pltpu.VMEM((CH, D), jnp.float32),
    ],
)
def _phase_a(idxr, emb2, dinv2, xp2, deg_s, idx_b, ones_b, dinv_v, row_v):
    cid = lax.axis_index("c")
    sid = lax.axis_index("s")

    _fill(dinv_v, CH, 0.0)
    for i in range(NCHN // NS):
        base = (sid + i * NS) * CH
        pltpu.sync_copy(dinv_v, deg_s.at[pl.ds(base, CH)])

    for g in range(CH // L):
        ones_b[0, pl.ds(g * L, L)] = jnp.full((L,), 1.0, jnp.float32)
    plsc.subcore_barrier()

    LAST = ECH - (NS - 1) * CPT

    @pl.when(sid < NS - 1)
    def _():
        pltpu.sync_copy(idxr.at[cid, pl.ds(sid * CPT, CPT), :], idx_b)

    @pl.when(sid == NS - 1)
    def _():
        pltpu.sync_copy(idxr.at[cid, pl.ds((NS - 1) * CPT, LAST), :],
                        idx_b.at[pl.ds(0, LAST), :])

    n = jnp.minimum(CPT, ECH - sid * CPT)

    def hist_body(c, carry):
        pltpu.sync_copy(ones_b.at[0], deg_s.at[idx_b.at[c]], add=True)
        return carry

    lax.fori_loop(0, n, hist_body, 0)
    plsc.subcore_barrier()

    for i in range(NCHN // NS):
        base = (sid + i * NS) * CH
        pltpu.sync_copy(deg_s.at[pl.ds(base, CH)], dinv_v)
        for g in range(CH // L):
            dinv_v[pl.ds(g * L, L)] = _vrsqrt(dinv_v[pl.ds(g * L, L)])
        pltpu.sync_copy(dinv_v, dinv2.at[cid, 0, pl.ds(base, CH)])
        pltpu.sync_copy(emb2.at[cid, pl.ds(base, CH), :], row_v)

        def row_body(r, rcarry):
            s = _splat_elem(dinv_v, r)
            for g in range(D // L):
                row_v[r, pl.ds(g * L, L)] = row_v[r, pl.ds(g * L, L)] * s
            return rcarry

        lax.fori_loop(0, CH, row_body, 0)
        pltpu.sync_copy(row_v, xp2.at[cid, pl.ds(base, CH), :])


@functools.partial(
    pl.kernel,
    out_type=jax.ShapeDtypeStruct((NC, NP, D), jnp.float32),
    mesh=_MESH,
    compiler_params=pltpu.CompilerParams(needs_layout_passes=False),
    scratch_types=[
        pltpu.VMEM_SHARED((NP, D), jnp.float32),
        pltpu.VMEM((4, CH), jnp.int32),
        pltpu.VMEM((4, CH), jnp.int32),
        pltpu.VMEM((2, CH, D), jnp.float32),
        pltpu.VMEM((CH,), jnp.float32),
        pltpu.SemaphoreType.DMA,
        pltpu.SemaphoreType.DMA,
        pltpu.SemaphoreType.DMA,
        pltpu.SemaphoreType.DMA,
        pltpu.SemaphoreType.DMA,
        pltpu.SemaphoreType.DMA,
    ],
)
def _phase_b(tbl2, idx2, emb2, dinv2, res2,
             acc_s, idxg_r, idxs_r, rows2, dinv_v,
             sem_a, sem_b, sem_i0, sem_i1, sem_i2, sem_i3):
    cid = lax.axis_index("c")
    sid = lax.axis_index("s")
    ocid = 1 - cid

    def zrow_body(r, carry):
        for g in range(D // L):
            rows2[0, r, pl.ds(g * L, L)] = jnp.full((L,), 0.0, jnp.float32)
        return carry

    lax.fori_loop(0, CH, zrow_body, 0)
    for i in range(NCHN // NS):
        base = (sid + i * NS) * CH
        pltpu.sync_copy(rows2.at[0], acc_s.at[pl.ds(base, CH), :])
    plsc.subcore_barrier()

    cbase = sid * CPT
    n = jnp.minimum(CPT, ECH - cbase)
    isems = (sem_i0, sem_i1, sem_i2, sem_i3)

    def istart(c, k):
        pltpu.async_copy(
            idx2.at[ocid, 0, pl.ds((cbase + c) * CH, CH)],
            idxg_r.at[k], isems[k])
        pltpu.async_copy(
            idx2.at[cid, 0, pl.ds((cbase + c) * CH, CH)],
            idxs_r.at[k], isems[k])

    def iwait(c, k):
        pltpu.make_async_copy(
            idx2.at[ocid, 0, pl.ds((cbase + c) * CH, CH)],
            idxg_r.at[k], isems[k]).wait()
        pltpu.make_async_copy(
            idx2.at[cid, 0, pl.ds((cbase + c) * CH, CH)],
            idxs_r.at[k], isems[k]).wait()

    def gstart(c, k, rs):
        pltpu.async_copy(
            tbl2.at[ocid].at[idxg_r.at[k]], rows2.at[rs],
            (sem_a, sem_b)[rs])

    def gwait(c, k, rs):
        pltpu.make_async_copy(
            tbl2.at[ocid].at[idxg_r.at[k]], rows2.at[rs],
            (sem_a, sem_b)[rs]).wait()

    for k in range(4):
        istart(k, k)
    iwait(0, 0)
    gstart(0, 0, 0)
    iwait(1, 1)
    gstart(1, 1, 1)

    def quad_body(i, carry):
        c0 = 4 * i
        for k in range(4):
            c = c0 + k
            rs = k % 2
            gwait(c, k, rs)
            pltpu.sync_copy(rows2.at[rs], acc_s.at[idxs_r.at[k]], add=True)

            @pl.when(c + 4 < n)
            def _():
                istart(c + 4, k)

            @pl.when(c + 2 < n)
            def _():
                iwait(c + 2, (k + 2) % 4)
                gstart(c + 2, (k + 2) % 4, rs)

        return carry

    lax.fori_loop(0, n // 4, quad_body, 0)
    plsc.subcore_barrier()

    for i in range(NCHN // NS):
        base = (sid + i * NS) * CH
        pltpu.sync_copy(acc_s.at[pl.ds(base, CH), :], rows2.at[0])
        pltpu.sync_copy(dinv2.at[cid, 0, pl.ds(base, CH)], dinv_v)
        pltpu.sync_copy(emb2.at[cid, pl.ds(base, CH), :], rows2.at[1])

        def row_body(r, rcarry):
            w = _splat_elem(dinv_v, r) * WSUM
            for g in range(D // L):
                rows2[1, r, pl.ds(g * L, L)] = (
                    rows2[1, r, pl.ds(g * L, L)]
                    + rows2[0, r, pl.ds(g * L, L)] * w
                )
            return rcarry

        lax.fori_loop(0, CH, row_body, 0)
        pltpu.sync_copy(rows2.at[1], res2.at[cid, pl.ds(base, CH), :])


@functools.partial(
    pl.kernel,
    out_type=jax.ShapeDtypeStruct((EL,), jnp.float32),
    mesh=_MESH,
    compiler_params=pltpu.CompilerParams(needs_layout_passes=False),
    scratch_types=[
        pltpu.VMEM((WPT, CH), jnp.int32),
        pltpu.VMEM((WPT, CH), jnp.int32),
        pltpu.VMEM((2, CH, D), jnp.float32),
        pltpu.VMEM((2, CH, D), jnp.float32),
        pltpu.VMEM((2 * CH,), jnp.float32),
        pltpu.VMEM((L * 17,), jnp.float32),
        pltpu.SemaphoreType.DMA,
        pltpu.SemaphoreType.DMA,
        pltpu.SemaphoreType.DMA,
        pltpu.SemaphoreType.DMA,
    ],
)
def _phase_c(res2, iup, imp, out,
             iu_v, im_v, u2, m2, out_v, p_buf, semu0, semu1, semm0, semm1):
    cid = lax.axis_index("c")
    sid = lax.axis_index("s")
    wid = sid * NC + cid
    pltpu.sync_copy(iup.at[pl.ds(wid * WPT, WPT), :], iu_v)
    pltpu.sync_copy(imp.at[pl.ds(wid * WPT, WPT), :], im_v)
    n = jnp.minimum(WPT, LCH - wid * WPT)
    semu = (semu0, semu1)
    semm = (semm0, semm1)
    lane = lax.iota(jnp.int32, L)

    def cstart(c, slot):
        pltpu.async_copy(res2.at[0].at[iu_v.at[c]], u2.at[slot], semu[slot])
        pltpu.async_copy(res2.at[1].at[im_v.at[c]], m2.at[slot], semm[slot])

    def cwait(c, slot):
        pltpu.make_async_copy(
            res2.at[0].at[iu_v.at[c]], u2.at[slot], semu[slot]).wait()
        pltpu.make_async_copy(
            res2.at[1].at[im_v.at[c]], m2.at[slot], semm[slot]).wait()

    cols = [lane * 17 + l for l in range(L)]

    def dots(slot, par):
        def group_body(g, carry):
            for ep in range(L):
                e = g * L + ep
                p = u2[slot, e, pl.ds(0, L)] * m2[slot, e, pl.ds(0, L)]
                for k in range(1, D // L):
                    p = p + (u2[slot, e, pl.ds(k * L, L)]
                             * m2[slot, e, pl.ds(k * L, L)])
                p_buf[pl.ds(ep * 17, L)] = p
            acc = plsc.load_gather(p_buf, [cols[0]])
            for l in range(1, L):
                acc = acc + plsc.load_gather(p_buf, [cols[l]])
            out_v[pl.ds(par * CH + g * L, L)] = acc
            return carry

        lax.fori_loop(0, CH // L, group_body, 0)

    cstart(0, 0)

    def pair_body(i, carry):
        c0 = 2 * i
        cstart(c0 + 1, 1)
        cwait(c0, 0)
        dots(0, 0)

        @pl.when(c0 + 2 < n)
        def _():
            cstart(c0 + 2, 0)

        cwait(c0 + 1, 1)
        dots(1, 1)
        pltpu.sync_copy(out_v, out.at[pl.ds((wid * WPT + c0) * CH, 2 * CH)])
        return carry

    lax.fori_loop(0, n // 2, pair_body, 0)


def kernel(user_node_id, movie_node_id, edge_index, edge_label_index,
           emb_user, emb_movie):
    del user_node_id, movie_node_id
    emb2 = jnp.zeros((NC, NP, D), jnp.float32)
    emb2 = emb2.at[0, :N, :].set(emb_user).at[1, :N, :].set(emb_movie)
    idx2 = edge_index.reshape(NC, 1, E)
    idxr = edge_index.reshape(NC, ECH, CH)
    dinv2, xp2 = _phase_a(idxr, emb2)
    res2 = _phase_b(xp2, idx2, emb2, dinv2)
    lab = jnp.zeros((2, ECHP * CH), jnp.int32)
    lab = lab.at[:, :EL].set(edge_label_index).reshape(2, ECHP, CH)
    res = _phase_c(res2, lab[1], lab[0])
    return res, res2[0, :N, :], res2[1, :N, :]


# speedup vs baseline: 1.1359x; 1.0377x over previous
import functools
import jax
import jax.numpy as jnp
from jax import lax
from jax.experimental import pallas as pl
from jax.experimental.pallas import tpu as pltpu
from jax.experimental.pallas import tpu_sc as plsc

N = 10000
D = 128
E = 320000
EL = 320000
NC = 2
NS = 16
L = 16
CH = 128
NP = 10240
NCHN = NP // CH
ECH = E // CH
LCH = EL // CH
ECHP = 2560
CPT = ECHP // NS
WPT = ECHP // (NC * NS)
NPA = 10112
NCHA = NPA // CH
WSUM = 5.0 / 6.0

_MESH = plsc.VectorSubcoreMesh(core_axis_name="c", subcore_axis_name="s")


def _vrsqrt(x):
    i = lax.bitcast_convert_type(x, jnp.int32)
    i = jnp.int32(0x5F3759DF) - lax.shift_right_logical(i, 1)
    y = lax.bitcast_convert_type(i, jnp.float32)
    for _ in range(3):
        y = y * (1.5 - 0.5 * x * y * y)
    return jnp.where(x > 0.0, y, 0.0)


def _splat_elem(ref, r):
    idx = jnp.broadcast_to(r, (L,)).astype(jnp.int32)
    return plsc.load_gather(ref, [idx])


def _fill(ref, n, value):
    for g in range(n // L):
        ref[pl.ds(g * L, L)] = jnp.full((L,), value, jnp.float32)


@functools.partial(
    pl.kernel,
    out_type=[
        jax.ShapeDtypeStruct((NC, 1, NP), jnp.float32),
        jax.ShapeDtypeStruct((NC, NP, D), jnp.float32),
    ],
    mesh=_MESH,
    compiler_params=pltpu.CompilerParams(needs_layout_passes=False, use_tc_tiling_on_sc=False),
    scratch_types=[
        pltpu.VMEM_SHARED((NP,), jnp.float32),
        pltpu.VMEM((CPT, CH), jnp.int32),
        pltpu.VMEM((1, CH), jnp.float32),
        pltpu.VMEM((CH,), jnp.float32),
        pltpu.VMEM((CH, D), jnp.float32),
    ],
)
def _phase_a(idxr, emb2, dinv2, xp2, deg_s, idx_b, ones_b, dinv_v, row_v):
    cid = lax.axis_index("c")
    sid = lax.axis_index("s")

    _fill(dinv_v, CH, 0.0)
    for i in range(NCHN // NS):
        base = (sid + i * NS) * CH
        pltpu.sync_copy(dinv_v, deg_s.at[pl.ds(base, CH)])

    for g in range(CH // L):
        ones_b[0, pl.ds(g * L, L)] = jnp.full((L,), 1.0, jnp.float32)
    plsc.subcore_barrier()

    LAST = ECH - (NS - 1) * CPT

    @pl.when(sid < NS - 1)
    def _():
        pltpu.sync_copy(idxr.at[cid, pl.ds(sid * CPT, CPT), :], idx_b)

    @pl.when(sid == NS - 1)
    def _():
        pltpu.sync_copy(idxr.at[cid, pl.ds((NS - 1) * CPT, LAST), :],
                        idx_b.at[pl.ds(0, LAST), :])

    n = jnp.minimum(CPT, ECH - sid * CPT)

    def hist_body(c, carry):
        pltpu.sync_copy(ones_b.at[0], deg_s.at[idx_b.at[c]], add=True)
        return carry

    lax.fori_loop(0, n, hist_body, 0)
    plsc.subcore_barrier()

    for i in range(NCHN // NS):
        base = (sid + i * NS) * CH
        pltpu.sync_copy(deg_s.at[pl.ds(base, CH)], dinv_v)
        for g in range(CH // L):
            dinv_v[pl.ds(g * L, L)] = _vrsqrt(dinv_v[pl.ds(g * L, L)])
        pltpu.sync_copy(dinv_v, dinv2.at[cid, 0, pl.ds(base, CH)])
        pltpu.sync_copy(emb2.at[cid, pl.ds(base, CH), :], row_v)

        def row_body(r, rcarry):
            s = _splat_elem(dinv_v, r)
            for g in range(D // L):
                row_v[r, pl.ds(g * L, L)] = row_v[r, pl.ds(g * L, L)] * s
            return rcarry

        lax.fori_loop(0, CH, row_body, 0)
        pltpu.sync_copy(row_v, xp2.at[cid, pl.ds(base, CH), :])


@functools.partial(
    pl.kernel,
    out_type=[
        jax.ShapeDtypeStruct((NC, NP, D), jnp.float32),
        jax.ShapeDtypeStruct((NC, NP, D // 2), jnp.float32),
    ],
    mesh=_MESH,
    compiler_params=pltpu.CompilerParams(needs_layout_passes=False, use_tc_tiling_on_sc=False),
    scratch_types=[
        pltpu.VMEM_SHARED((NPA, D), jnp.float32),
        pltpu.VMEM((4, CH), jnp.int32),
        pltpu.VMEM((4, CH), jnp.int32),
        pltpu.VMEM((2, CH, D), jnp.float32),
        pltpu.VMEM((CH // 2, D // 2), jnp.float32),
        pltpu.VMEM((CH,), jnp.float32),
        pltpu.SemaphoreType.DMA,
        pltpu.SemaphoreType.DMA,
        pltpu.SemaphoreType.DMA,
        pltpu.SemaphoreType.DMA,
        pltpu.SemaphoreType.DMA,
        pltpu.SemaphoreType.DMA,
    ],
)
def _phase_b(tbl2, idx2, emb2, dinv2, res2, resb2,
             acc_s, idxg_r, idxs_r, rows2, brow_v, dinv_v,
             sem_a, sem_b, sem_i0, sem_i1, sem_i2, sem_i3):
    cid = lax.axis_index("c")
    sid = lax.axis_index("s")
    ocid = 1 - cid

    def zrow_body(r, carry):
        for g in range(D // L):
            rows2[0, r, pl.ds(g * L, L)] = jnp.full((L,), 0.0, jnp.float32)
        return carry

    lax.fori_loop(0, CH, zrow_body, 0)
    nza = (NCHA - sid + NS - 1) // NS

    def zero_body(i, carry):
        base = (sid + i * NS) * CH
        pltpu.sync_copy(rows2.at[0], acc_s.at[pl.ds(base, CH), :])
        return carry

    lax.fori_loop(0, nza, zero_body, 0)
    plsc.subcore_barrier()

    cbase = sid * CPT
    n = jnp.minimum(CPT, ECH - cbase)
    isems = (sem_i0, sem_i1, sem_i2, sem_i3)

    def istart(c, k):
        pltpu.async_copy(
            idx2.at[ocid, 0, pl.ds((cbase + c) * CH, CH)],
            idxg_r.at[k], isems[k])
        pltpu.async_copy(
            idx2.at[cid, 0, pl.ds((cbase + c) * CH, CH)],
            idxs_r.at[k], isems[k])

    def iwait(c, k):
        pltpu.make_async_copy(
            idx2.at[ocid, 0, pl.ds((cbase + c) * CH, CH)],
            idxg_r.at[k], isems[k]).wait()
        pltpu.make_async_copy(
            idx2.at[cid, 0, pl.ds((cbase + c) * CH, CH)],
            idxs_r.at[k], isems[k]).wait()

    def gstart(c, k, rs):
        pltpu.async_copy(
            tbl2.at[ocid].at[idxg_r.at[k]], rows2.at[rs],
            (sem_a, sem_b)[rs])

    def gwait(c, k, rs):
        pltpu.make_async_copy(
            tbl2.at[ocid].at[idxg_r.at[k]], rows2.at[rs],
            (sem_a, sem_b)[rs]).wait()

    for k in range(4):
        istart(k, k)
    iwait(0, 0)
    gstart(0, 0, 0)
    iwait(1, 1)
    gstart(1, 1, 1)

    def quad_body(i, carry):
        c0 = 4 * i
        for k in range(4):
            c = c0 + k
            rs = k % 2
            gwait(c, k, rs)
            pltpu.sync_copy(rows2.at[rs], acc_s.at[idxs_r.at[k]], add=True)

            @pl.when(c + 4 < n)
            def _():
                istart(c + 4, k)

            @pl.when(c + 2 < n)
            def _():
                iwait(c + 2, (k + 2) % 4)
                gstart(c + 2, (k + 2) % 4, rs)

        return carry

    lax.fori_loop(0, n // 4, quad_body, 0)
    plsc.subcore_barrier()

    def epi_body(i, carry):
        base = (sid + i * NS) * CH
        pltpu.sync_copy(acc_s.at[pl.ds(base, CH), :], rows2.at[0])
        pltpu.sync_copy(dinv2.at[cid, 0, pl.ds(base, CH)], dinv_v)
        pltpu.sync_copy(emb2.at[cid, pl.ds(base, CH), :], rows2.at[1])

        for h in range(2):
            def row_body(r, rcarry):
                rr = h * (CH // 2) + r
                w = _splat_elem(dinv_v, rr) * WSUM
                for g in range(D // (2 * L)):
                    va = (rows2[1, rr, pl.ds(2 * g * L, L)]
                          + rows2[0, rr, pl.ds(2 * g * L, L)] * w)
                    vb = (rows2[1, rr, pl.ds((2 * g + 1) * L, L)]
                          + rows2[0, rr, pl.ds((2 * g + 1) * L, L)] * w)
                    rows2[1, rr, pl.ds(2 * g * L, L)] = va
                    rows2[1, rr, pl.ds((2 * g + 1) * L, L)] = vb
                    brow_v[r, pl.ds(g * L, L)] = plsc.bitcast(
                        plsc.pack(va, vb,
                                  format=plsc.PackFormat.INTERLEAVED),
                        jnp.float32)
                return rcarry

            lax.fori_loop(0, CH // 2, row_body, 0)
            pltpu.sync_copy(
                brow_v,
                resb2.at[cid, pl.ds(base + h * (CH // 2), CH // 2), :])
        pltpu.sync_copy(rows2.at[1], res2.at[cid, pl.ds(base, CH), :])
        return carry

    lax.fori_loop(0, nza, epi_body, 0)


@functools.partial(
    pl.kernel,
    out_type=jax.ShapeDtypeStruct((EL,), jnp.float32),
    mesh=_MESH,
    compiler_params=pltpu.CompilerParams(needs_layout_passes=False, use_tc_tiling_on_sc=False),
    scratch_types=[
        pltpu.VMEM((WPT, CH), jnp.int32),
        pltpu.VMEM((WPT, CH), jnp.int32),
        pltpu.VMEM((2, CH, D // 2), jnp.float32),
        pltpu.VMEM((2, CH, D // 2), jnp.float32),
        pltpu.VMEM((2 * CH,), jnp.float32),
        pltpu.VMEM((L * 17,), jnp.float32),
        pltpu.SemaphoreType.DMA,
        pltpu.SemaphoreType.DMA,
        pltpu.SemaphoreType.DMA,
        pltpu.SemaphoreType.DMA,
    ],
)
def _phase_c(res2, iup, imp, out,
             iu_v, im_v, u2, m2, out_v, p_buf, semu0, semu1, semm0, semm1):
    cid = lax.axis_index("c")
    sid = lax.axis_index("s")
    wid = sid * NC + cid
    pltpu.sync_copy(iup.at[pl.ds(wid * WPT, WPT), :], iu_v)
    pltpu.sync_copy(imp.at[pl.ds(wid * WPT, WPT), :], im_v)
    n = jnp.minimum(WPT, LCH - wid * WPT)
    semu = (semu0, semu1)
    semm = (semm0, semm1)
    lane = lax.iota(jnp.int32, L)

    def cstart(c, slot):
        pltpu.async_copy(res2.at[0].at[iu_v.at[c]], u2.at[slot], semu[slot])
        pltpu.async_copy(res2.at[1].at[im_v.at[c]], m2.at[slot], semm[slot])

    def cwait(c, slot):
        pltpu.make_async_copy(
            res2.at[0].at[iu_v.at[c]], u2.at[slot], semu[slot]).wait()
        pltpu.make_async_copy(
            res2.at[1].at[im_v.at[c]], m2.at[slot], semm[slot]).wait()

    cols = [lane * 17 + l for l in range(L)]

    def dots(slot, par):
        def group_body(g, carry):
            for ep in range(L):
                e = g * L + ep
                p = None
                for k in range(D // (2 * L)):
                    ub = plsc.bitcast(u2[slot, e, pl.ds(k * L, L)],
                                      jnp.bfloat16)
                    mb = plsc.bitcast(m2[slot, e, pl.ds(k * L, L)],
                                      jnp.bfloat16)
                    pa, pb = plsc.unpack(
                        ub * mb, format=plsc.PackFormat.INTERLEAVED)
                    q = pa + pb
                    p = q if p is None else p + q
                p_buf[pl.ds(ep * 17, L)] = p
            acc = plsc.load_gather(p_buf, [cols[0]])
            for l in range(1, L):
                acc = acc + plsc.load_gather(p_buf, [cols[l]])
            out_v[pl.ds(par * CH + g * L, L)] = acc
            return carry

        lax.fori_loop(0, CH // L, group_body, 0)

    cstart(0, 0)

    def pair_body(i, carry):
        c0 = 2 * i
        cstart(c0 + 1, 1)
        cwait(c0, 0)
        dots(0, 0)

        @pl.when(c0 + 2 < n)
        def _():
            cstart(c0 + 2, 0)

        cwait(c0 + 1, 1)
        dots(1, 1)
        pltpu.sync_copy(out_v, out.at[pl.ds((wid * WPT + c0) * CH, 2 * CH)])
        return carry

    lax.fori_loop(0, n // 2, pair_body, 0)


def kernel(user_node_id, movie_node_id, edge_index, edge_label_index,
           emb_user, emb_movie):
    del user_node_id, movie_node_id
    emb2 = jnp.zeros((NC, NP, D), jnp.float32)
    emb2 = emb2.at[0, :N, :].set(emb_user).at[1, :N, :].set(emb_movie)
    idx2 = edge_index.reshape(NC, 1, E)
    idxr = edge_index.reshape(NC, ECH, CH)
    dinv2, xp2 = _phase_a(idxr, emb2)
    res2, resb2 = _phase_b(xp2, idx2, emb2, dinv2)
    lab = jnp.zeros((2, ECHP * CH), jnp.int32)
    lab = lab.at[:, :EL].set(edge_label_index).reshape(2, ECHP, CH)
    res = _phase_c(resb2, lab[1], lab[0])
    return res, res2[0, :N, :], res2[1, :N, :]


# speedup vs baseline: 1.1390x; 1.0027x over previous
import functools
import jax
import jax.numpy as jnp
from jax import lax
from jax.experimental import pallas as pl
from jax.experimental.pallas import tpu as pltpu
from jax.experimental.pallas import tpu_sc as plsc

N = 10000
D = 128
E = 320000
EL = 320000
NC = 2
NS = 16
L = 16
CH = 128
NP = 10240
NCHN = NP // CH
ECH = E // CH
LCH = EL // CH
ECHP = 2560
CPT = ECHP // NS
WPT = ECHP // (NC * NS)
NPA = 10112
NCHA = NPA // CH
WSUM = 5.0 / 6.0

_MESH = plsc.VectorSubcoreMesh(core_axis_name="c", subcore_axis_name="s")


def _vrsqrt(x):
    i = lax.bitcast_convert_type(x, jnp.int32)
    i = jnp.int32(0x5F3759DF) - lax.shift_right_logical(i, 1)
    y = lax.bitcast_convert_type(i, jnp.float32)
    for _ in range(3):
        y = y * (1.5 - 0.5 * x * y * y)
    return jnp.where(x > 0.0, y, 0.0)


def _splat_elem(ref, r):
    idx = jnp.broadcast_to(r, (L,)).astype(jnp.int32)
    return plsc.load_gather(ref, [idx])


def _fill(ref, n, value):
    for g in range(n // L):
        ref[pl.ds(g * L, L)] = jnp.full((L,), value, jnp.float32)


@functools.partial(
    pl.kernel,
    out_type=[
        jax.ShapeDtypeStruct((NC, 1, NP), jnp.float32),
        jax.ShapeDtypeStruct((NC, NP, D), jnp.float32),
    ],
    mesh=_MESH,
    compiler_params=pltpu.CompilerParams(needs_layout_passes=False, use_tc_tiling_on_sc=False),
    scratch_types=[
        pltpu.VMEM_SHARED((NP,), jnp.float32),
        pltpu.VMEM((CPT, CH), jnp.int32),
        pltpu.VMEM((1, CH), jnp.float32),
        pltpu.VMEM((CH,), jnp.float32),
        pltpu.VMEM((CH, D), jnp.float32),
    ],
)
def _phase_a(idxr, emb2, dinv2, xp2, deg_s, idx_b, ones_b, dinv_v, row_v):
    cid = lax.axis_index("c")
    sid = lax.axis_index("s")

    _fill(dinv_v, CH, 0.0)
    for i in range(NCHN // NS):
        base = (sid + i * NS) * CH
        pltpu.sync_copy(dinv_v, deg_s.at[pl.ds(base, CH)])

    for g in range(CH // L):
        ones_b[0, pl.ds(g * L, L)] = jnp.full((L,), 1.0, jnp.float32)
    plsc.subcore_barrier()

    LAST = ECH - (NS - 1) * CPT

    @pl.when(sid < NS - 1)
    def _():
        pltpu.sync_copy(idxr.at[cid, pl.ds(sid * CPT, CPT), :], idx_b)

    @pl.when(sid == NS - 1)
    def _():
        pltpu.sync_copy(idxr.at[cid, pl.ds((NS - 1) * CPT, LAST), :],
                        idx_b.at[pl.ds(0, LAST), :])

    n = jnp.minimum(CPT, ECH - sid * CPT)

    def hist_body(c, carry):
        pltpu.sync_copy(ones_b.at[0], deg_s.at[idx_b.at[c]], add=True)
        return carry

    lax.fori_loop(0, n, hist_body, 0)
    plsc.subcore_barrier()

    for i in range(NCHN // NS):
        base = (sid + i * NS) * CH
        pltpu.sync_copy(deg_s.at[pl.ds(base, CH)], dinv_v)
        for g in range(CH // L):
            dinv_v[pl.ds(g * L, L)] = _vrsqrt(dinv_v[pl.ds(g * L, L)])
        pltpu.sync_copy(dinv_v, dinv2.at[cid, 0, pl.ds(base, CH)])
        pltpu.sync_copy(emb2.at[cid, pl.ds(base, CH), :], row_v)

        def row_body(r, rcarry):
            s = _splat_elem(dinv_v, r)
            for g in range(D // L):
                row_v[r, pl.ds(g * L, L)] = row_v[r, pl.ds(g * L, L)] * s
            return rcarry

        lax.fori_loop(0, CH, row_body, 0)
        pltpu.sync_copy(row_v, xp2.at[cid, pl.ds(base, CH), :])


@functools.partial(
    pl.kernel,
    out_type=[
        jax.ShapeDtypeStruct((NC, NP, D), jnp.float32),
        jax.ShapeDtypeStruct((NC, NP, D // 2), jnp.float32),
    ],
    mesh=_MESH,
    compiler_params=pltpu.CompilerParams(needs_layout_passes=False, use_tc_tiling_on_sc=False),
    scratch_types=[
        pltpu.VMEM_SHARED((NPA, D), jnp.float32),
        pltpu.VMEM((4, CH), jnp.int32),
        pltpu.VMEM((4, CH), jnp.int32),
        pltpu.VMEM((2, CH, D), jnp.float32),
        pltpu.VMEM((CH // 2, D // 2), jnp.float32),
        pltpu.VMEM((CH,), jnp.float32),
        pltpu.SemaphoreType.DMA,
        pltpu.SemaphoreType.DMA,
        pltpu.SemaphoreType.DMA,
        pltpu.SemaphoreType.DMA,
        pltpu.SemaphoreType.DMA,
        pltpu.SemaphoreType.DMA,
    ],
)
def _phase_b(tbl2, idx2, emb2, dinv2, res2, resb2,
             acc_s, idxg_r, idxs_r, rows2, brow_v, dinv_v,
             sem_a, sem_b, sem_i0, sem_i1, sem_i2, sem_i3):
    cid = lax.axis_index("c")
    sid = lax.axis_index("s")
    ocid = 1 - cid

    def zrow_body(r, carry):
        for g in range(D // L):
            rows2[0, r, pl.ds(g * L, L)] = jnp.full((L,), 0.0, jnp.float32)
        return carry

    lax.fori_loop(0, CH, zrow_body, 0)
    nza = (NCHA - sid + NS - 1) // NS

    def zero_body(i, carry):
        base = (sid + i * NS) * CH
        pltpu.sync_copy(rows2.at[0], acc_s.at[pl.ds(base, CH), :])
        return carry

    lax.fori_loop(0, nza, zero_body, 0)
    plsc.subcore_barrier()

    cbase = sid * CPT
    n = jnp.minimum(CPT, ECH - cbase)
    isems = (sem_i0, sem_i1, sem_i2, sem_i3)

    def istart(c, k):
        pltpu.async_copy(
            idx2.at[ocid, 0, pl.ds((cbase + c) * CH, CH)],
            idxg_r.at[k], isems[k])
        pltpu.async_copy(
            idx2.at[cid, 0, pl.ds((cbase + c) * CH, CH)],
            idxs_r.at[k], isems[k])

    def iwait(c, k):
        pltpu.make_async_copy(
            idx2.at[ocid, 0, pl.ds((cbase + c) * CH, CH)],
            idxg_r.at[k], isems[k]).wait()
        pltpu.make_async_copy(
            idx2.at[cid, 0, pl.ds((cbase + c) * CH, CH)],
            idxs_r.at[k], isems[k]).wait()

    def gstart(c, k, rs):
        pltpu.async_copy(
            tbl2.at[ocid].at[idxg_r.at[k]], rows2.at[rs],
            (sem_a, sem_b)[rs])

    def gwait(c, k, rs):
        pltpu.make_async_copy(
            tbl2.at[ocid].at[idxg_r.at[k]], rows2.at[rs],
            (sem_a, sem_b)[rs]).wait()

    for k in range(4):
        istart(k, k)
    iwait(0, 0)
    gstart(0, 0, 0)
    iwait(1, 1)
    gstart(1, 1, 1)

    def quad_body(i, carry):
        c0 = 4 * i
        for k in range(4):
            c = c0 + k
            rs = k % 2
            gwait(c, k, rs)
            pltpu.sync_copy(rows2.at[rs], acc_s.at[idxs_r.at[k]], add=True)

            @pl.when(c + 4 < n)
            def _():
                istart(c + 4, k)

            @pl.when(c + 2 < n)
            def _():
                iwait(c + 2, (k + 2) % 4)
                gstart(c + 2, (k + 2) % 4, rs)

        return carry

    lax.fori_loop(0, n // 4, quad_body, 0)
    plsc.subcore_barrier()

    def epi_body(i, carry):
        base = (sid + i * NS) * CH
        pltpu.sync_copy(acc_s.at[pl.ds(base, CH), :], rows2.at[0])
        pltpu.sync_copy(dinv2.at[cid, 0, pl.ds(base, CH)], dinv_v)
        pltpu.sync_copy(emb2.at[cid, pl.ds(base, CH), :], rows2.at[1])

        for h in range(2):
            def row_body(r, rcarry):
                rr = h * (CH // 2) + r
                w = _splat_elem(dinv_v, rr) * WSUM
                for g in range(D // (2 * L)):
                    va = (rows2[1, rr, pl.ds(2 * g * L, L)]
                          + rows2[0, rr, pl.ds(2 * g * L, L)] * w)
                    vb = (rows2[1, rr, pl.ds((2 * g + 1) * L, L)]
                          + rows2[0, rr, pl.ds((2 * g + 1) * L, L)] * w)
                    rows2[1, rr, pl.ds(2 * g * L, L)] = va
                    rows2[1, rr, pl.ds((2 * g + 1) * L, L)] = vb
                    brow_v[r, pl.ds(g * L, L)] = plsc.bitcast(
                        plsc.pack(va, vb,
                                  format=plsc.PackFormat.INTERLEAVED),
                        jnp.float32)
                return rcarry

            lax.fori_loop(0, CH // 2, row_body, 0)
            pltpu.sync_copy(
                brow_v,
                resb2.at[cid, pl.ds(base + h * (CH // 2), CH // 2), :])
        pltpu.sync_copy(rows2.at[1], res2.at[cid, pl.ds(base, CH), :])
        return carry

    lax.fori_loop(0, nza, epi_body, 0)


@functools.partial(
    pl.kernel,
    out_type=jax.ShapeDtypeStruct((EL,), jnp.float32),
    mesh=_MESH,
    compiler_params=pltpu.CompilerParams(needs_layout_passes=False, use_tc_tiling_on_sc=False),
    scratch_types=[
        pltpu.VMEM((WPT, CH), jnp.int32),
        pltpu.VMEM((WPT, CH), jnp.int32),
        pltpu.VMEM((2, CH, D // 2), jnp.float32),
        pltpu.VMEM((2, CH, D // 2), jnp.float32),
        pltpu.VMEM((2, 2 * CH), jnp.float32),
        pltpu.VMEM((L * 17,), jnp.float32),
        pltpu.SemaphoreType.DMA,
        pltpu.SemaphoreType.DMA,
        pltpu.SemaphoreType.DMA,
        pltpu.SemaphoreType.DMA,
        pltpu.SemaphoreType.DMA,
        pltpu.SemaphoreType.DMA,
    ],
)
def _phase_c(res2, iup, imp, out,
             iu_v, im_v, u2, m2, out_v, p_buf, semu0, semu1, semm0, semm1,
             semo0, semo1):
    cid = lax.axis_index("c")
    sid = lax.axis_index("s")
    wid = sid * NC + cid
    pltpu.sync_copy(iup.at[pl.ds(wid * WPT, WPT), :], iu_v)
    pltpu.sync_copy(imp.at[pl.ds(wid * WPT, WPT), :], im_v)
    n = jnp.minimum(WPT, LCH - wid * WPT)
    semu = (semu0, semu1)
    semm = (semm0, semm1)
    lane = lax.iota(jnp.int32, L)

    def cstart(c, slot):
        pltpu.async_copy(res2.at[0].at[iu_v.at[c]], u2.at[slot], semu[slot])
        pltpu.async_copy(res2.at[1].at[im_v.at[c]], m2.at[slot], semm[slot])

    def cwait(c, slot):
        pltpu.make_async_copy(
            res2.at[0].at[iu_v.at[c]], u2.at[slot], semu[slot]).wait()
        pltpu.make_async_copy(
            res2.at[1].at[im_v.at[c]], m2.at[slot], semm[slot]).wait()

    cols = [lane * 17 + l for l in range(L)]

    semo = (semo0, semo1)

    def dots(slot, par, ov):
        def group_body(g, carry):
            for ep in range(L):
                e = g * L + ep
                p = None
                for k in range(D // (2 * L)):
                    ub = plsc.bitcast(u2[slot, e, pl.ds(k * L, L)],
                                      jnp.bfloat16)
                    mb = plsc.bitcast(m2[slot, e, pl.ds(k * L, L)],
                                      jnp.bfloat16)
                    pa, pb = plsc.unpack(
                        ub * mb, format=plsc.PackFormat.INTERLEAVED)
                    q = pa + pb
                    p = q if p is None else p + q
                p_buf[pl.ds(ep * 17, L)] = p
            acc = plsc.load_gather(p_buf, [cols[0]])
            for l in range(1, L):
                acc = acc + plsc.load_gather(p_buf, [cols[l]])
            out_v[ov, pl.ds(par * CH + g * L, L)] = acc
            return carry

        lax.fori_loop(0, CH // L, group_body, 0)

    def odesc(p, sp):
        return pltpu.make_async_copy(
            out_v.at[sp],
            out.at[pl.ds((wid * WPT + 2 * p) * CH, 2 * CH)], semo[sp])

    cstart(0, 0)

    def dpair_body(j, carry):
        for sp in range(2):
            p = 2 * j + sp
            c0 = 2 * p

            @pl.when(p >= 2)
            def _():
                odesc(p - 2, sp).wait()

            cstart(c0 + 1, 1)
            cwait(c0, 0)
            dots(0, 0, sp)

            @pl.when(c0 + 2 < n)
            def _():
                cstart(c0 + 2, 0)

            cwait(c0 + 1, 1)
            dots(1, 1, sp)
            pltpu.async_copy(
                out_v.at[sp],
                out.at[pl.ds((wid * WPT + c0) * CH, 2 * CH)], semo[sp])
        return carry

    npair = n // 2
    lax.fori_loop(0, npair // 2, dpair_body, 0)
    odesc(npair - 2, 0).wait()
    odesc(npair - 1, 1).wait()


def kernel(user_node_id, movie_node_id, edge_index, edge_label_index,
           emb_user, emb_movie):
    del user_node_id, movie_node_id
    emb2 = jnp.zeros((NC, NP, D), jnp.float32)
    emb2 = emb2.at[0, :N, :].set(emb_user).at[1, :N, :].set(emb_movie)
    idx2 = edge_index.reshape(NC, 1, E)
    idxr = edge_index.reshape(NC, ECH, CH)
    dinv2, xp2 = _phase_a(idxr, emb2)
    res2, resb2 = _phase_b(xp2, idx2, emb2, dinv2)
    lab = jnp.zeros((2, ECHP * CH), jnp.int32)
    lab = lab.at[:, :EL].set(edge_label_index).reshape(2, ECHP, CH)
    res = _phase_c(resb2, lab[1], lab[0])
    return res, res2[0, :N, :], res2[1, :N, :]
